# baseline (speedup 1.0000x reference)
import functools

import jax
import jax.numpy as jnp
from jax import lax
from jax.experimental import pallas as pl
from jax.experimental.pallas import tpu as pltpu
from jax.experimental.pallas import tpu_sc as plsc

_NC = 2
_NS = 16
_L = 16
_W = 128
_NEG = float(jnp.finfo(jnp.float32).min)


@functools.lru_cache(maxsize=None)
def _build(B, S, H, NI):
    NW = _NC * _NS
    P = B * NI
    SUB = H // _W
    RPC = _L // SUB
    SUBLOG = SUB.bit_length() - 1
    assert H % _W == 0 and _L % SUB == 0 and SUB == 1 << SUBLOG
    PPW = (P + NW - 1) // NW

    mesh = plsc.VectorSubcoreMesh(core_axis_name="c", subcore_axis_name="s")

    @functools.partial(
        pl.kernel,
        mesh=mesh,
        out_type=jax.ShapeDtypeStruct((P, SUB, _W), jnp.float32),
        scratch_types=[
            pltpu.VMEM((_L, _W), jnp.float32),
            pltpu.VMEM((SUB, _W), jnp.float32),
            pltpu.VMEM((2 * P + _L,), jnp.int32),
            pltpu.SemaphoreType.DMA,
        ],
    )
    def sc_kernel(hid_hbm, se_hbm, miss_hbm, out_hbm,
                  rows_v, acc_v, se_v, sem):
        wid = lax.axis_index("s") * _NC + lax.axis_index("c")

        def do_pair(p):
            b = p // NI
            i = p % NI

            pltpu.sync_copy(se_hbm, se_v.at[pl.ds(0, 2 * P)])
            s = se_v[pl.ds(p, _L)][0]
            e = se_v[pl.ds(P + p, _L)][0]
            valid = jnp.logical_and(s >= 0, e >= 0)
            cs = jnp.clip(s, 0, S)
            ce = jnp.clip(e, 0, S)
            ln = ce - cs
            base = (b * S + cs) * SUB

            for j in range(SUB):
                for k in range(_W // _L):
                    acc_v[j, pl.ds(k * _L, _L)] = jnp.full(
                        (_L,), _NEG, jnp.float32)

            nchunks = jnp.maximum(-(-ln // RPC), 0)

            def chunk_body(c, carry):
                lanes = lax.iota(jnp.int32, _L)
                jvec = lanes & (SUB - 1)
                rvec = lanes >> SUBLOG
                row = jnp.minimum(c * RPC + rvec, ln - 1)
                idx = base + row * SUB + jvec
                pltpu.async_copy(hid_hbm.at[idx], rows_v, sem).wait()
                for j in range(SUB):
                    for k in range(_W // _L):
                        sl = pl.ds(k * _L, _L)
                        m = rows_v[j, sl]
                        for r in range(1, RPC):
                            m = jnp.maximum(m, rows_v[r * SUB + j, sl])
                        acc_v[j, sl] = jnp.maximum(acc_v[j, sl], m)
                return carry

            lax.fori_loop(0, nchunks, chunk_body, jnp.int32(0))

            @pl.when(jnp.logical_not(valid))
            def _():
                pltpu.sync_copy(miss_hbm.at[i], acc_v)

            pltpu.sync_copy(acc_v, out_hbm.at[p])

        for t in range(PPW):
            p = wid + t * NW
            if P % NW == 0:
                do_pair(p)
            else:
                pl.when(p < P)(lambda: do_pair(p))

    return sc_kernel


def kernel(hidden_state, start_indices, end_indices, missing_embeddings):
    B, S, H = hidden_state.shape
    NI = start_indices.shape[1]
    sc = _build(B, S, H, NI)
    se = jnp.concatenate(
        [start_indices.reshape(B * NI), end_indices.reshape(B * NI)])
    out = sc(
        hidden_state.reshape(B * S * (H // _W), _W),
        se,
        missing_embeddings.reshape(NI, H // _W, _W),
        )
    return out.reshape(B, NI * H)
